# baseline (device time: 2645054 ns/iter reference)
import jax
import jax.numpy as jnp
from jax import lax
from jax.experimental import pallas as pl
from jax.experimental.pallas import tpu as pltpu

N_Y = 4
V_SHARD = 8192
K = 16


def kernel(ids, E):
    my_y = lax.axis_index("y")
    local = ids - my_y * V_SHARD
    t = ids.shape[0]
    d = E.shape[1]
    ck = t // K

    def body(idx_ref, e_hbm, out_ref, lin_hbm, rin_hbm,
             lst, rst, lfwd, rfwd,
             lrecv_sems, rrecv_sems, lsend_sems, rsend_sems,
             lst_sems, rst_sems, gather_sem):
        my_x = lax.axis_index("x")
        my_y = lax.axis_index("y")
        my_z = lax.axis_index("z")
        has_left = my_y > 0
        has_right = my_y < N_Y - 1

        barrier_sem = pltpu.get_barrier_semaphore()

        @pl.when(has_left)
        def _():
            pl.semaphore_signal(
                barrier_sem, inc=1,
                device_id=(my_x, my_y - 1, my_z),
                device_id_type=pl.DeviceIdType.MESH,
            )

        @pl.when(has_right)
        def _():
            pl.semaphore_signal(
                barrier_sem, inc=1,
                device_id=(my_x, my_y + 1, my_z),
                device_id_type=pl.DeviceIdType.MESH,
            )

        @pl.when(has_left & has_right)
        def _():
            pl.semaphore_wait(barrier_sem, 2)

        @pl.when(~(has_left & has_right))
        def _():
            pl.semaphore_wait(barrier_sem, 1)

        def scan_chunk(k):

            def gather_one(tok, count):
                v = idx_ref[tok]
                valid = (v >= 0) & (v < V_SHARD)

                @pl.when(valid)
                def _():
                    pltpu.make_async_copy(
                        e_hbm.at[pl.ds(v, 1), :],
                        out_ref.at[pl.ds(tok, 1), :],
                        gather_sem,
                    ).start()

                return count + valid.astype(jnp.int32)

            return lax.fori_loop(k * ck, (k + 1) * ck, gather_one, jnp.int32(0))

        def drain_gather(count):
            def drain_one(_, carry):
                pltpu.make_async_copy(
                    e_hbm.at[pl.ds(0, 1), :],
                    out_ref.at[pl.ds(0, 1), :],
                    gather_sem,
                ).wait()
                return carry

            lax.fori_loop(0, count, drain_one, jnp.int32(0))

        def stream_rdma(fwd_slot, in_hbm, rows, send_sem, recv_sem, dst_y):
            return pltpu.make_async_remote_copy(
                src_ref=fwd_slot,
                dst_ref=in_hbm.at[rows, :],
                send_sem=send_sem,
                recv_sem=recv_sem,
                device_id=(my_x, dst_y, my_z),
                device_id_type=pl.DeviceIdType.MESH,
            )

        for k in range(K):
            rows = pl.ds(k * ck, ck)
            sl = k % 2

            out_ref[rows, :] = jnp.zeros((ck, d), out_ref.dtype)
            drain_gather(scan_chunk(k))

            @pl.when(has_left)
            def _():
                stream_rdma(lfwd.at[sl], lin_hbm, rows,
                            lsend_sems.at[sl], lrecv_sems.at[k], my_y).wait_recv()
                cp = pltpu.make_async_copy(lin_hbm.at[rows, :], lst.at[sl],
                                           lst_sems.at[sl])
                cp.start()
                cp.wait()

            @pl.when(has_right)
            def _():
                if k >= 2:
                    stream_rdma(lfwd.at[sl], lin_hbm, rows,
                                lsend_sems.at[sl], lrecv_sems.at[k], my_y).wait_send()

                @pl.when(has_left)
                def _():
                    lfwd[sl] = out_ref[rows, :] + lst[sl]

                @pl.when(~has_left)
                def _():
                    lfwd[sl] = out_ref[rows, :]

                stream_rdma(lfwd.at[sl], lin_hbm, rows,
                            lsend_sems.at[sl], lrecv_sems.at[k], my_y + 1).start()

            @pl.when(has_right)
            def _():
                stream_rdma(rfwd.at[sl], rin_hbm, rows,
                            rsend_sems.at[sl], rrecv_sems.at[k], my_y).wait_recv()
                cp = pltpu.make_async_copy(rin_hbm.at[rows, :], rst.at[sl],
                                           rst_sems.at[sl])
                cp.start()
                cp.wait()

            @pl.when(has_left)
            def _():
                if k >= 2:
                    stream_rdma(rfwd.at[sl], rin_hbm, rows,
                                rsend_sems.at[sl], rrecv_sems.at[k], my_y).wait_send()

                @pl.when(has_right)
                def _():
                    rfwd[sl] = out_ref[rows, :] + rst[sl]

                @pl.when(~has_right)
                def _():
                    rfwd[sl] = out_ref[rows, :]

                stream_rdma(rfwd.at[sl], rin_hbm, rows,
                            rsend_sems.at[sl], rrecv_sems.at[k], my_y - 1).start()

            @pl.when(has_left)
            def _():
                out_ref[rows, :] += lst[sl]

            @pl.when(has_right)
            def _():
                out_ref[rows, :] += rst[sl]

        for sl in range(2):
            @pl.when(has_right)
            def _():
                stream_rdma(lfwd.at[sl], lin_hbm, pl.ds(0, ck),
                            lsend_sems.at[sl], lrecv_sems.at[0], my_y).wait_send()

            @pl.when(has_left)
            def _():
                stream_rdma(rfwd.at[sl], rin_hbm, pl.ds(0, ck),
                            rsend_sems.at[sl], rrecv_sems.at[0], my_y).wait_send()

    out = pl.pallas_call(
        body,
        out_shape=[
            jax.ShapeDtypeStruct((t, d), E.dtype),
            jax.ShapeDtypeStruct((t, d), E.dtype),
            jax.ShapeDtypeStruct((t, d), E.dtype),
        ],
        in_specs=[
            pl.BlockSpec(memory_space=pltpu.SMEM),
            pl.BlockSpec(memory_space=pl.ANY),
        ],
        out_specs=[
            pl.BlockSpec(memory_space=pltpu.VMEM),
            pl.BlockSpec(memory_space=pl.ANY),
            pl.BlockSpec(memory_space=pl.ANY),
        ],
        scratch_shapes=[
            pltpu.VMEM((2, ck, d), E.dtype),
            pltpu.VMEM((2, ck, d), E.dtype),
            pltpu.VMEM((2, ck, d), E.dtype),
            pltpu.VMEM((2, ck, d), E.dtype),
            pltpu.SemaphoreType.DMA((K,)),
            pltpu.SemaphoreType.DMA((K,)),
            pltpu.SemaphoreType.DMA((2,)),
            pltpu.SemaphoreType.DMA((2,)),
            pltpu.SemaphoreType.DMA((2,)),
            pltpu.SemaphoreType.DMA((2,)),
            pltpu.SemaphoreType.DMA,
        ],
        compiler_params=pltpu.CompilerParams(
            collective_id=0,
            vmem_limit_bytes=60 * 1024 * 1024,
        ),
    )(local, E)
    return out[0]


# device time: 1110650 ns/iter; 2.3815x vs baseline; 2.3815x over previous
import jax
import jax.numpy as jnp
from jax import lax
from jax.experimental import pallas as pl
from jax.experimental.pallas import tpu as pltpu

N_Y = 4
V_SHARD = 8192
K = 16


def kernel(ids, E):
    my_y = lax.axis_index("y")
    local = ids - my_y * V_SHARD
    t = ids.shape[0]
    d = E.shape[1]
    ck = t // K

    def body(idx_ref, e_hbm, out_ref, lin_hbm, rin_hbm,
             lst, rst, lfwd, rfwd,
             lrecv_sems, rrecv_sems, lsend_sems, rsend_sems,
             lst_sems, rst_sems, gather_sem):
        my_x = lax.axis_index("x")
        my_y = lax.axis_index("y")
        my_z = lax.axis_index("z")
        has_left = my_y > 0
        has_right = my_y < N_Y - 1

        barrier_sem = pltpu.get_barrier_semaphore()

        @pl.when(has_left)
        def _():
            pl.semaphore_signal(
                barrier_sem, inc=1,
                device_id=(my_x, my_y - 1, my_z),
                device_id_type=pl.DeviceIdType.MESH,
            )

        @pl.when(has_right)
        def _():
            pl.semaphore_signal(
                barrier_sem, inc=1,
                device_id=(my_x, my_y + 1, my_z),
                device_id_type=pl.DeviceIdType.MESH,
            )

        @pl.when(has_left & has_right)
        def _():
            pl.semaphore_wait(barrier_sem, 2)

        @pl.when(~(has_left & has_right))
        def _():
            pl.semaphore_wait(barrier_sem, 1)

        def prep_chunk(k):
            rows = pl.ds(k * ck, ck)
            out_ref[rows, :] = jnp.zeros((ck, d), out_ref.dtype)

            def gather_one(tok, count):
                v = idx_ref[tok]
                valid = (v >= 0) & (v < V_SHARD)

                @pl.when(valid)
                def _():
                    pltpu.make_async_copy(
                        e_hbm.at[pl.ds(v, 1), :],
                        out_ref.at[pl.ds(tok, 1), :],
                        gather_sem,
                    ).start()

                return count + valid.astype(jnp.int32)

            count = lax.fori_loop(k * ck, (k + 1) * ck, gather_one, jnp.int32(0))

            def drain_one(_, carry):
                pltpu.make_async_copy(
                    e_hbm.at[pl.ds(0, 1), :],
                    out_ref.at[pl.ds(0, 1), :],
                    gather_sem,
                ).wait()
                return carry

            lax.fori_loop(0, count, drain_one, jnp.int32(0))

        def stream_rdma(fwd_slot, in_hbm, rows, send_sem, recv_sem, dst_y):
            return pltpu.make_async_remote_copy(
                src_ref=fwd_slot,
                dst_ref=in_hbm.at[rows, :],
                send_sem=send_sem,
                recv_sem=recv_sem,
                device_id=(my_x, dst_y, my_z),
                device_id_type=pl.DeviceIdType.MESH,
            )

        def recv_stream(k, in_hbm, stg, recv_sems, stg_sems):
            sl = k % 2
            rows = pl.ds(k * ck, ck)
            stream_rdma(stg.at[sl], in_hbm, rows,
                        lsend_sems.at[sl], recv_sems.at[k], my_y).wait_recv()
            cp = pltpu.make_async_copy(in_hbm.at[rows, :], stg.at[sl],
                                       stg_sems.at[sl])
            cp.start()
            cp.wait()

        def send_stream(k, fwd, in_hbm, send_sems, recv_sems, dst_y, payload):
            sl = k % 2
            rows = pl.ds(k * ck, ck)
            if k >= 2:
                stream_rdma(fwd.at[sl], in_hbm, rows,
                            send_sems.at[sl], recv_sems.at[k], my_y).wait_send()
            fwd[sl] = payload
            stream_rdma(fwd.at[sl], in_hbm, rows,
                        send_sems.at[sl], recv_sems.at[k], dst_y).start()

        def drain_sends(fwd, in_hbm, send_sems, recv_sems):
            for sl in range(2):
                stream_rdma(fwd.at[sl], in_hbm, pl.ds(0, ck),
                            send_sems.at[sl], recv_sems.at[0], my_y).wait_send()

        @pl.when(~has_left)
        def _():
            for k in range(K):
                rows = pl.ds(k * ck, ck)
                prep_chunk(k)
                send_stream(k, lfwd, lin_hbm, lsend_sems, lrecv_sems,
                            my_y + 1, out_ref[rows, :])
            for k in range(K):
                rows = pl.ds(k * ck, ck)
                recv_stream(k, rin_hbm, rst, rrecv_sems, rst_sems)
                out_ref[rows, :] += rst[k % 2]
            drain_sends(lfwd, lin_hbm, lsend_sems, lrecv_sems)

        @pl.when(~has_right)
        def _():
            for k in range(K):
                rows = pl.ds(k * ck, ck)
                prep_chunk(k)
                send_stream(k, rfwd, rin_hbm, rsend_sems, rrecv_sems,
                            my_y - 1, out_ref[rows, :])
            for k in range(K):
                rows = pl.ds(k * ck, ck)
                recv_stream(k, lin_hbm, lst, lrecv_sems, lst_sems)
                out_ref[rows, :] += lst[k % 2]
            drain_sends(rfwd, rin_hbm, rsend_sems, rrecv_sems)

        @pl.when(has_left & has_right)
        def _():
            for k in range(K):
                rows = pl.ds(k * ck, ck)
                sl = k % 2
                prep_chunk(k)
                recv_stream(k, lin_hbm, lst, lrecv_sems, lst_sems)
                send_stream(k, lfwd, lin_hbm, lsend_sems, lrecv_sems,
                            my_y + 1, out_ref[rows, :] + lst[sl])
                recv_stream(k, rin_hbm, rst, rrecv_sems, rst_sems)
                send_stream(k, rfwd, rin_hbm, rsend_sems, rrecv_sems,
                            my_y - 1, out_ref[rows, :] + rst[sl])
                out_ref[rows, :] += lst[sl] + rst[sl]
            drain_sends(lfwd, lin_hbm, lsend_sems, lrecv_sems)
            drain_sends(rfwd, rin_hbm, rsend_sems, rrecv_sems)

    out = pl.pallas_call(
        body,
        out_shape=[
            jax.ShapeDtypeStruct((t, d), E.dtype),
            jax.ShapeDtypeStruct((t, d), E.dtype),
            jax.ShapeDtypeStruct((t, d), E.dtype),
        ],
        in_specs=[
            pl.BlockSpec(memory_space=pltpu.SMEM),
            pl.BlockSpec(memory_space=pl.ANY),
        ],
        out_specs=[
            pl.BlockSpec(memory_space=pltpu.VMEM),
            pl.BlockSpec(memory_space=pl.ANY),
            pl.BlockSpec(memory_space=pl.ANY),
        ],
        scratch_shapes=[
            pltpu.VMEM((2, ck, d), E.dtype),
            pltpu.VMEM((2, ck, d), E.dtype),
            pltpu.VMEM((2, ck, d), E.dtype),
            pltpu.VMEM((2, ck, d), E.dtype),
            pltpu.SemaphoreType.DMA((K,)),
            pltpu.SemaphoreType.DMA((K,)),
            pltpu.SemaphoreType.DMA((2,)),
            pltpu.SemaphoreType.DMA((2,)),
            pltpu.SemaphoreType.DMA((2,)),
            pltpu.SemaphoreType.DMA((2,)),
            pltpu.SemaphoreType.DMA,
        ],
        compiler_params=pltpu.CompilerParams(
            collective_id=0,
            vmem_limit_bytes=60 * 1024 * 1024,
        ),
    )(local, E)
    return out[0]


# device time: 491769 ns/iter; 5.3787x vs baseline; 2.2585x over previous
import jax
import jax.numpy as jnp
from jax import lax
from jax.experimental import pallas as pl
from jax.experimental.pallas import tpu as pltpu

N_Y = 4
V_SHARD = 8192
K = 16


def kernel(ids, E):
    my_y = lax.axis_index("y")
    local = ids - my_y * V_SHARD
    t = ids.shape[0]
    d = E.shape[1]
    ck = t // K

    def body(idx_ref, e_hbm, out_ref, lin_hbm, rin_hbm,
             lst, rst, lfwd, rfwd,
             lrecv_sems, rrecv_sems, lsend_sems, rsend_sems,
             lst_sems, rst_sems, gather_sem):
        my_x = lax.axis_index("x")
        my_y = lax.axis_index("y")
        my_z = lax.axis_index("z")
        has_left = my_y > 0
        has_right = my_y < N_Y - 1

        barrier_sem = pltpu.get_barrier_semaphore()

        @pl.when(has_left)
        def _():
            pl.semaphore_signal(
                barrier_sem, inc=1,
                device_id=(my_x, my_y - 1, my_z),
                device_id_type=pl.DeviceIdType.MESH,
            )

        @pl.when(has_right)
        def _():
            pl.semaphore_signal(
                barrier_sem, inc=1,
                device_id=(my_x, my_y + 1, my_z),
                device_id_type=pl.DeviceIdType.MESH,
            )

        @pl.when(has_left & has_right)
        def _():
            pl.semaphore_wait(barrier_sem, 2)

        @pl.when(~(has_left & has_right))
        def _():
            pl.semaphore_wait(barrier_sem, 1)

        def prep_chunk(k):
            rows = pl.ds(k * ck, ck)
            out_ref[rows, :] = jnp.zeros((ck, d), out_ref.dtype)

            def gather_one(tok, count):
                v = idx_ref[tok]
                valid = (v >= 0) & (v < V_SHARD)

                @pl.when(valid)
                def _():
                    pltpu.make_async_copy(
                        e_hbm.at[pl.ds(v, 1), :],
                        out_ref.at[pl.ds(tok, 1), :],
                        gather_sem,
                    ).start()

                return count + valid.astype(jnp.int32)

            count = lax.fori_loop(k * ck, (k + 1) * ck, gather_one, jnp.int32(0))

            def drain_one(_, carry):
                pltpu.make_async_copy(
                    e_hbm.at[pl.ds(0, 1), :],
                    out_ref.at[pl.ds(0, 1), :],
                    gather_sem,
                ).wait()
                return carry

            lax.fori_loop(0, count, drain_one, jnp.int32(0))

        def stream_rdma(fwd_slot, in_hbm, rows, send_sem, recv_sem, dst_y):
            return pltpu.make_async_remote_copy(
                src_ref=fwd_slot,
                dst_ref=in_hbm.at[rows, :],
                send_sem=send_sem,
                recv_sem=recv_sem,
                device_id=(my_x, dst_y, my_z),
                device_id_type=pl.DeviceIdType.MESH,
            )

        def recv_stream(k, in_hbm, stg, recv_sems, stg_sems):
            sl = k % 2
            rows = pl.ds(k * ck, ck)
            stream_rdma(stg.at[sl], in_hbm, rows,
                        lsend_sems.at[sl], recv_sems.at[k], my_y).wait_recv()
            cp = pltpu.make_async_copy(in_hbm.at[rows, :], stg.at[sl],
                                       stg_sems.at[sl])
            cp.start()
            cp.wait()

        def send_stream(k, fwd, in_hbm, send_sems, recv_sems, dst_y, payload):
            sl = k % 2
            rows = pl.ds(k * ck, ck)
            if k >= 2:
                stream_rdma(fwd.at[sl], in_hbm, rows,
                            send_sems.at[sl], recv_sems.at[k], my_y).wait_send()
            fwd[sl] = payload
            stream_rdma(fwd.at[sl], in_hbm, rows,
                        send_sems.at[sl], recv_sems.at[k], dst_y).start()

        def drain_sends(fwd, in_hbm, send_sems, recv_sems):
            for sl in range(2):
                stream_rdma(fwd.at[sl], in_hbm, pl.ds(0, ck),
                            send_sems.at[sl], recv_sems.at[0], my_y).wait_send()

        @pl.when(~has_left)
        def _():
            for k in range(K):
                rows = pl.ds(k * ck, ck)
                prep_chunk(k)
                send_stream(k, lfwd, lin_hbm, lsend_sems, lrecv_sems,
                            my_y + 1, out_ref[rows, :])
            for k in range(K):
                rows = pl.ds(k * ck, ck)
                recv_stream(k, rin_hbm, rst, rrecv_sems, rst_sems)
                out_ref[rows, :] += rst[k % 2]
            drain_sends(lfwd, lin_hbm, lsend_sems, lrecv_sems)

        @pl.when(~has_right)
        def _():
            for k in range(K):
                rows = pl.ds(k * ck, ck)
                prep_chunk(k)
                send_stream(k, rfwd, rin_hbm, rsend_sems, rrecv_sems,
                            my_y - 1, out_ref[rows, :])
            for k in range(K):
                rows = pl.ds(k * ck, ck)
                recv_stream(k, lin_hbm, lst, lrecv_sems, lst_sems)
                out_ref[rows, :] += lst[k % 2]
            drain_sends(rfwd, rin_hbm, rsend_sems, rrecv_sems)

        L = (lin_hbm, lst, lfwd, lrecv_sems, lsend_sems, lst_sems)
        R = (rin_hbm, rst, rfwd, rrecv_sems, rsend_sems, rst_sems)

        def middle_role(near, far, near_dst, far_dst):
            n_hbm, n_st, n_fwd, n_recv, n_send, n_stg = near
            f_hbm, f_st, f_fwd, f_recv, f_send, f_stg = far
            for k in range(K + 1):
                if k < K:
                    rows = pl.ds(k * ck, ck)
                    prep_chunk(k)
                    recv_stream(k, n_hbm, n_st, n_recv, n_stg)
                    send_stream(k, n_fwd, n_hbm, n_send, n_recv, near_dst,
                                out_ref[rows, :] + n_st[k % 2])
                if k >= 1:
                    c = k - 1
                    crows = pl.ds(c * ck, ck)
                    recv_stream(c, f_hbm, f_st, f_recv, f_stg)
                    send_stream(c, f_fwd, f_hbm, f_send, f_recv, far_dst,
                                out_ref[crows, :] + f_st[c % 2])
                    out_ref[crows, :] += n_st[c % 2] + f_st[c % 2]
            drain_sends(n_fwd, n_hbm, n_send, n_recv)
            drain_sends(f_fwd, f_hbm, f_send, f_recv)

        @pl.when(my_y == 1)
        def _():
            middle_role(L, R, my_y + 1, my_y - 1)

        @pl.when(my_y == 2)
        def _():
            middle_role(R, L, my_y - 1, my_y + 1)

    out = pl.pallas_call(
        body,
        out_shape=[
            jax.ShapeDtypeStruct((t, d), E.dtype),
            jax.ShapeDtypeStruct((t, d), E.dtype),
            jax.ShapeDtypeStruct((t, d), E.dtype),
        ],
        in_specs=[
            pl.BlockSpec(memory_space=pltpu.SMEM),
            pl.BlockSpec(memory_space=pl.ANY),
        ],
        out_specs=[
            pl.BlockSpec(memory_space=pltpu.VMEM),
            pl.BlockSpec(memory_space=pl.ANY),
            pl.BlockSpec(memory_space=pl.ANY),
        ],
        scratch_shapes=[
            pltpu.VMEM((2, ck, d), E.dtype),
            pltpu.VMEM((2, ck, d), E.dtype),
            pltpu.VMEM((2, ck, d), E.dtype),
            pltpu.VMEM((2, ck, d), E.dtype),
            pltpu.SemaphoreType.DMA((K,)),
            pltpu.SemaphoreType.DMA((K,)),
            pltpu.SemaphoreType.DMA((2,)),
            pltpu.SemaphoreType.DMA((2,)),
            pltpu.SemaphoreType.DMA((2,)),
            pltpu.SemaphoreType.DMA((2,)),
            pltpu.SemaphoreType.DMA,
        ],
        compiler_params=pltpu.CompilerParams(
            collective_id=0,
            vmem_limit_bytes=60 * 1024 * 1024,
        ),
    )(local, E)
    return out[0]


# device time: 485913 ns/iter; 5.4435x vs baseline; 1.0121x over previous
import jax
import jax.numpy as jnp
from jax import lax
from jax.experimental import pallas as pl
from jax.experimental.pallas import tpu as pltpu

N_Y = 4
V_SHARD = 8192
K = 16
S = 4
LAG = 4


def kernel(ids, E):
    my_y = lax.axis_index("y")
    local = ids - my_y * V_SHARD
    t = ids.shape[0]
    d = E.shape[1]
    ck = t // K

    def body(idx_ref, e_hbm, out_ref,
             lnd_l, lnd_r, lfwd, rfwd,
             lrecv_sems, rrecv_sems, lsend_sems, rsend_sems,
             credit_l, credit_r, gather_sem):
        my_x = lax.axis_index("x")
        my_y = lax.axis_index("y")
        my_z = lax.axis_index("z")
        has_left = my_y > 0
        has_right = my_y < N_Y - 1

        barrier_sem = pltpu.get_barrier_semaphore()

        @pl.when(has_left)
        def _():
            pl.semaphore_signal(
                barrier_sem, inc=1,
                device_id=(my_x, my_y - 1, my_z),
                device_id_type=pl.DeviceIdType.MESH,
            )

        @pl.when(has_right)
        def _():
            pl.semaphore_signal(
                barrier_sem, inc=1,
                device_id=(my_x, my_y + 1, my_z),
                device_id_type=pl.DeviceIdType.MESH,
            )

        @pl.when(has_left & has_right)
        def _():
            pl.semaphore_wait(barrier_sem, 2)

        @pl.when(~(has_left & has_right))
        def _():
            pl.semaphore_wait(barrier_sem, 1)

        def prep_chunk(k):
            rows = pl.ds(k * ck, ck)
            out_ref[rows, :] = jnp.zeros((ck, d), out_ref.dtype)

            def gather_one(tok, count):
                v = idx_ref[tok]
                valid = (v >= 0) & (v < V_SHARD)

                @pl.when(valid)
                def _():
                    pltpu.make_async_copy(
                        e_hbm.at[pl.ds(v, 1), :],
                        out_ref.at[pl.ds(tok, 1), :],
                        gather_sem,
                    ).start()

                return count + valid.astype(jnp.int32)

            count = lax.fori_loop(k * ck, (k + 1) * ck, gather_one, jnp.int32(0))

            def drain_one(_, carry):
                pltpu.make_async_copy(
                    e_hbm.at[pl.ds(0, 1), :],
                    out_ref.at[pl.ds(0, 1), :],
                    gather_sem,
                ).wait()
                return carry

            lax.fori_loop(0, count, drain_one, jnp.int32(0))

        def stream_rdma(fwd_slot, lnd, k, send_sem, recv_sem, dst_y):
            return pltpu.make_async_remote_copy(
                src_ref=fwd_slot,
                dst_ref=lnd.at[k % S],
                send_sem=send_sem,
                recv_sem=recv_sem,
                device_id=(my_x, dst_y, my_z),
                device_id_type=pl.DeviceIdType.MESH,
            )

        def recv_wait(k, lnd, fwd, recv_sems):
            stream_rdma(fwd.at[0], lnd, k,
                        lsend_sems.at[0], recv_sems.at[k], my_y).wait_recv()

        def send_stream(k, fwd, lnd, send_sems, recv_sems, credit, dst_y, payload):
            sl = k % 2
            if k >= S:
                pl.semaphore_wait(credit, 1)
            if k >= 2:
                stream_rdma(fwd.at[sl], lnd, k,
                            send_sems.at[sl], recv_sems.at[k], my_y).wait_send()
            fwd[sl] = payload
            stream_rdma(fwd.at[sl], lnd, k,
                        send_sems.at[sl], recv_sems.at[k], dst_y).start()

        def give_credit(credit, src_y):
            pl.semaphore_signal(
                credit, inc=1,
                device_id=(my_x, src_y, my_z),
                device_id_type=pl.DeviceIdType.MESH,
            )

        def drain_sends(fwd, lnd, send_sems, recv_sems):
            for sl in range(2):
                stream_rdma(fwd.at[sl], lnd, sl,
                            send_sems.at[sl], recv_sems.at[0], my_y).wait_send()

        @pl.when(~has_left)
        def _():
            for k in range(K + LAG):
                if k < K:
                    rows = pl.ds(k * ck, ck)
                    prep_chunk(k)
                    send_stream(k, lfwd, lnd_l, lsend_sems, lrecv_sems,
                                credit_l, my_y + 1, out_ref[rows, :])
                if k >= LAG:
                    c = k - LAG
                    crows = pl.ds(c * ck, ck)
                    recv_wait(c, lnd_r, rfwd, rrecv_sems)
                    out_ref[crows, :] += lnd_r[c % S]
                    give_credit(credit_r, my_y + 1)
            drain_sends(lfwd, lnd_l, lsend_sems, lrecv_sems)
            pl.semaphore_wait(credit_l, S)

        @pl.when(~has_right)
        def _():
            for k in range(K + LAG):
                if k < K:
                    rows = pl.ds(k * ck, ck)
                    prep_chunk(k)
                    send_stream(k, rfwd, lnd_r, rsend_sems, rrecv_sems,
                                credit_r, my_y - 1, out_ref[rows, :])
                if k >= LAG:
                    c = k - LAG
                    crows = pl.ds(c * ck, ck)
                    recv_wait(c, lnd_l, lfwd, lrecv_sems)
                    out_ref[crows, :] += lnd_l[c % S]
                    give_credit(credit_l, my_y - 1)
            drain_sends(rfwd, lnd_r, rsend_sems, rrecv_sems)
            pl.semaphore_wait(credit_r, S)

        Lc = (lnd_l, lfwd, lrecv_sems, lsend_sems, credit_l)
        Rc = (lnd_r, rfwd, rrecv_sems, rsend_sems, credit_r)

        def middle_role(near, far, near_dst, far_dst):
            n_lnd, n_fwd, n_recv, n_send, n_cred = near
            f_lnd, f_fwd, f_recv, f_send, f_cred = far
            for k in range(K + 1):
                if k < K:
                    rows = pl.ds(k * ck, ck)
                    prep_chunk(k)
                    recv_wait(k, n_lnd, n_fwd, n_recv)
                    send_stream(k, n_fwd, n_lnd, n_send, n_recv, n_cred,
                                near_dst, out_ref[rows, :] + n_lnd[k % S])
                if k >= 1:
                    c = k - 1
                    crows = pl.ds(c * ck, ck)
                    recv_wait(c, f_lnd, f_fwd, f_recv)
                    send_stream(c, f_fwd, f_lnd, f_send, f_recv, f_cred,
                                far_dst, out_ref[crows, :] + f_lnd[c % S])
                    out_ref[crows, :] += n_lnd[c % S] + f_lnd[c % S]
                    give_credit(n_cred, far_dst)
                    give_credit(f_cred, near_dst)
            drain_sends(n_fwd, n_lnd, n_send, n_recv)
            drain_sends(f_fwd, f_lnd, f_send, f_recv)
            pl.semaphore_wait(n_cred, S)
            pl.semaphore_wait(f_cred, S)

        @pl.when(my_y == 1)
        def _():
            middle_role(Lc, Rc, my_y + 1, my_y - 1)

        @pl.when(my_y == 2)
        def _():
            middle_role(Rc, Lc, my_y - 1, my_y + 1)

    return pl.pallas_call(
        body,
        out_shape=jax.ShapeDtypeStruct((t, d), E.dtype),
        in_specs=[
            pl.BlockSpec(memory_space=pltpu.SMEM),
            pl.BlockSpec(memory_space=pl.ANY),
        ],
        out_specs=pl.BlockSpec(memory_space=pltpu.VMEM),
        scratch_shapes=[
            pltpu.VMEM((S, ck, d), E.dtype),
            pltpu.VMEM((S, ck, d), E.dtype),
            pltpu.VMEM((2, ck, d), E.dtype),
            pltpu.VMEM((2, ck, d), E.dtype),
            pltpu.SemaphoreType.DMA((K,)),
            pltpu.SemaphoreType.DMA((K,)),
            pltpu.SemaphoreType.DMA((2,)),
            pltpu.SemaphoreType.DMA((2,)),
            pltpu.SemaphoreType.REGULAR,
            pltpu.SemaphoreType.REGULAR,
            pltpu.SemaphoreType.DMA,
        ],
        compiler_params=pltpu.CompilerParams(
            collective_id=0,
            vmem_limit_bytes=60 * 1024 * 1024,
        ),
    )(local, E)
